# baseline (speedup 1.0000x reference)
import functools

import jax
import jax.numpy as jnp
import numpy as np
from jax import lax
from jax.experimental import pallas as pl
from jax.experimental.pallas import tpu as pltpu
from jax.experimental.pallas import tpu_sc as plsc

_FEATURE_NUM = 128
_ACTIVE_LO = 4
_N_ACT = 124
_MIN_MAX = {4: (0.0, 1000.0), 5: (-10.0, 10.0), 6: (0.0, 1.0), 7: (0.0, 255.0)}

_SC_NC = 2
_SC_NS = 16
_NW = _SC_NC * _SC_NS
_CH = 128


def _col_consts():
    cmin = np.zeros((1, _FEATURE_NUM), np.float32)
    cmax = np.ones((1, _FEATURE_NUM), np.float32)
    for c, (lo, hi) in _MIN_MAX.items():
        cmin[0, c] = lo
        cmax[0, c] = hi
    inv = 1.0 / (cmax - cmin)
    return cmin, inv


def _vec_consts():
    cmin128, inv128 = _col_consts()
    return (cmin128[0, _ACTIVE_LO:_ACTIVE_LO + 16].copy(),
            inv128[0, _ACTIVE_LO:_ACTIVE_LO + 16].copy())


def _tc_body(x_ref, cmin_ref, inv_ref, ad_ref):
    x = x_ref[...]
    n = (x - cmin_ref[...]) * inv_ref[...]
    n = jnp.clip(n, 0.0, 1.0)
    bad = (x == -1.0) | jnp.isnan(x)
    n = jnp.where(bad, 0.0, n)
    na = n[:, _ACTIVE_LO:]
    ad_ref[0] = na
    ad_ref[1] = na * na
    ad_ref[2] = jnp.sqrt(na)


def _tc_call(features):
    rows = features.shape[0]
    block = 4096
    cmin, inv = _col_consts()
    return pl.pallas_call(
        _tc_body,
        grid=(rows // block,),
        in_specs=[
            pl.BlockSpec((block, _FEATURE_NUM), lambda i: (i, 0)),
            pl.BlockSpec((1, _FEATURE_NUM), lambda i: (0, 0)),
            pl.BlockSpec((1, _FEATURE_NUM), lambda i: (0, 0)),
        ],
        out_specs=pl.BlockSpec((3, block, _N_ACT), lambda i: (0, i, 0)),
        out_shape=jax.ShapeDtypeStruct((3, rows, _N_ACT), jnp.float32),
    )(features, jnp.asarray(cmin), jnp.asarray(inv))


def _sc_norm_body(feat_hbm, cmin_hbm, inv_hbm, out_hbm, cmin_v, inv_v, in_v,
                  out_v, sem_in0, sem_in1, sem_out0, sem_out1):
    per_w = feat_hbm.shape[0] // _NW
    nch = per_w // _CH
    wid = lax.axis_index("s") * _SC_NC + lax.axis_index("c")
    base = wid * per_w
    pltpu.sync_copy(cmin_hbm, cmin_v)
    pltpu.sync_copy(inv_hbm, inv_v)
    cmin = cmin_v[...]
    inv = inv_v[...]
    sems_in = (sem_in0, sem_in1)
    sems_out = (sem_out0, sem_out1)
    copies_in = {}
    copies_out = {}
    copies_in[0] = pltpu.async_copy(
        feat_hbm.at[pl.ds(base, _CH)], in_v.at[0], sems_in[0])
    for g in range(nch):
        b = g % 2
        b2 = (g + 1) % 2
        if g + 1 < nch:
            copies_in[g + 1] = pltpu.async_copy(
                feat_hbm.at[pl.ds(base + (g + 1) * _CH, _CH)], in_v.at[b2],
                sems_in[b2])
        copies_in[g].wait()
        if g - 2 >= 0:
            copies_out[g - 2].wait()

        def row(r, carry):
            x0 = in_v[b, r, pl.ds(4, 16)]
            n0 = (x0 - cmin) * inv
            n0 = jnp.minimum(jnp.maximum(n0, 0.0), 1.0)
            out_v[b, r, pl.ds(0, 16)] = n0
            for t in range(1, 8):
                off = 16 * t + 4 if t < 7 else 112
                out_v[b, r, pl.ds(off - 4, 16)] = in_v[b, r, pl.ds(off, 16)]
            return carry

        lax.fori_loop(0, _CH, row, jnp.int32(0))
        copies_out[g] = pltpu.async_copy(
            out_v.at[b], out_hbm.at[pl.ds(base + g * _CH, _CH)], sems_out[b])
    if nch - 2 >= 0:
        copies_out[nch - 2].wait()
    copies_out[nch - 1].wait()


def _sc_call(features):
    rows = features.shape[0]
    cmin16, inv16 = _vec_consts()
    mesh = plsc.VectorSubcoreMesh(core_axis_name="c", subcore_axis_name="s")
    fn = functools.partial(
        pl.kernel,
        out_type=jax.ShapeDtypeStruct((rows, _N_ACT), jnp.float32),
        mesh=mesh,
        scratch_types=[
            pltpu.VMEM((16,), jnp.float32),
            pltpu.VMEM((16,), jnp.float32),
            pltpu.VMEM((2, _CH, _FEATURE_NUM), jnp.float32),
            pltpu.VMEM((2, _CH, _N_ACT), jnp.float32),
            pltpu.SemaphoreType.DMA,
            pltpu.SemaphoreType.DMA,
            pltpu.SemaphoreType.DMA,
            pltpu.SemaphoreType.DMA,
        ],
    )(_sc_norm_body)
    return fn(features, jnp.asarray(cmin16), jnp.asarray(inv16))


@jax.jit
def kernel(features):
    ad_planes = _tc_call(features)
    norm = _sc_call(features)
    return jnp.transpose(ad_planes, (1, 2, 0)), norm


# speedup vs baseline: 2.2149x; 2.2149x over previous
import jax
import jax.numpy as jnp
import numpy as np
from jax.experimental import pallas as pl

_FEATURE_NUM = 128
_ACTIVE_LO = 4
_N_ACT = _FEATURE_NUM - _ACTIVE_LO
_MIN_MAX = {4: (0.0, 1000.0), 5: (-10.0, 10.0), 6: (0.0, 1.0), 7: (0.0, 255.0)}


def _col_consts():
    cmin = np.zeros((1, _FEATURE_NUM), np.float32)
    cmax = np.ones((1, _FEATURE_NUM), np.float32)
    for c, (lo, hi) in _MIN_MAX.items():
        cmin[0, c] = lo
        cmax[0, c] = hi
    inv = 1.0 / (cmax - cmin)
    return cmin, inv


def _tc_body(x_ref, cmin_ref, inv_ref, ad_ref, norm_ref):
    x = x_ref[...]
    n = (x - cmin_ref[...]) * inv_ref[...]
    n = jnp.clip(n, 0.0, 1.0)
    bad = (x == -1.0) | jnp.isnan(x)
    n = jnp.where(bad, 0.0, n)
    na = n[:, _ACTIVE_LO:]
    norm_ref[...] = na
    ad_ref[0] = na
    ad_ref[1] = na * na
    ad_ref[2] = jnp.sqrt(na)


@jax.jit
def kernel(features):
    rows = features.shape[0]
    block = 4096
    grid = rows // block
    cmin, inv = _col_consts()
    ad_planes, norm = pl.pallas_call(
        _tc_body,
        grid=(grid,),
        in_specs=[
            pl.BlockSpec((block, _FEATURE_NUM), lambda i: (i, 0)),
            pl.BlockSpec((1, _FEATURE_NUM), lambda i: (0, 0)),
            pl.BlockSpec((1, _FEATURE_NUM), lambda i: (0, 0)),
        ],
        out_specs=[
            pl.BlockSpec((3, block, _N_ACT), lambda i: (0, i, 0)),
            pl.BlockSpec((block, _N_ACT), lambda i: (i, 0)),
        ],
        out_shape=[
            jax.ShapeDtypeStruct((3, rows, _N_ACT), jnp.float32),
            jax.ShapeDtypeStruct((rows, _N_ACT), jnp.float32),
        ],
    )(features, jnp.asarray(cmin), jnp.asarray(inv))
    return jnp.transpose(ad_planes, (1, 2, 0)), norm
